# Initial kernel scaffold; baseline (speedup 1.0000x reference)
import jax
import jax.numpy as jnp
from jax.experimental import pallas as pl


def kernel(edge_index, edge_type, emb, W_rel1, W_root1, b1, W_rel2, W_root2, b2):
    raise NotImplementedError("write your pallas kernel here")


# speedup vs baseline: 6.8769x; 6.8769x over previous
import dataclasses
import functools

import jax
import jax.numpy as jnp
from jax import lax
from jax.experimental import pallas as pl
from jax.experimental.pallas import tpu as pltpu
from jax.experimental.pallas import tpu_sc as plsc

N = 10000
R = 8
DIM = 128
E = 320000
G = 128
NWIN = E // G
NC = 2
NS = 16
NW = NC * NS
KMAX = (NWIN + NW - 1) // NW
NR = N * R
ROWS_PER_SUB = 640
ACC_ROWS = NS * ROWS_PER_SUB
CROWS_PER_SUB = NR // NS
NB = 2000

_mesh = plsc.VectorSubcoreMesh(core_axis_name="c", subcore_axis_name="s")

_sc_params = pltpu.CompilerParams()
if "needs_layout_passes" in pltpu.CompilerParams.__dataclass_fields__:
    _sc_params = dataclasses.replace(_sc_params, needs_layout_passes=False)


def _h_body(x_ref, w_ref, h_ref):
    h_ref[0] = jnp.dot(x_ref[...], w_ref[0], preferred_element_type=jnp.float32)


def _h_all(x, w_rel):
    return pl.pallas_call(
        _h_body,
        grid=(N // NB, R),
        in_specs=[
            pl.BlockSpec((NB, DIM), lambda i, r: (i, 0)),
            pl.BlockSpec((1, DIM, DIM), lambda i, r: (r, 0, 0)),
        ],
        out_specs=pl.BlockSpec((1, NB, DIM), lambda i, r: (r, i, 0)),
        out_shape=jax.ShapeDtypeStruct((R, N, DIM), jnp.float32),
    )(x, w_rel)


def _inv_body(c_ref, o_ref):
    o_ref[...] = 1.0 / jnp.maximum(c_ref[0] + c_ref[1], 1.0)


def _inv_counts(cnt2):
    return pl.pallas_call(
        _inv_body,
        grid=(N // NB,),
        in_specs=[pl.BlockSpec((2, NB, DIM), lambda i: (0, i, 0))],
        out_specs=pl.BlockSpec((NB, DIM), lambda i: (i, 0)),
        out_shape=jax.ShapeDtypeStruct((N, DIM), jnp.float32),
    )(cnt2)


def _combine_body(x_ref, w_ref, b_ref, a_ref, o_ref):
    y = jnp.dot(x_ref[...], w_ref[...], preferred_element_type=jnp.float32)
    o_ref[...] = jnp.maximum(y + b_ref[...] + a_ref[0] + a_ref[1], 0.0)


def _combine(x, w_root, b2d, acc):
    return pl.pallas_call(
        _combine_body,
        grid=(N // NB,),
        in_specs=[
            pl.BlockSpec((NB, DIM), lambda i: (i, 0)),
            pl.BlockSpec((DIM, DIM), lambda i: (0, 0)),
            pl.BlockSpec((1, DIM), lambda i: (0, 0)),
            pl.BlockSpec((2, NB, DIM), lambda i: (0, i, 0)),
        ],
        out_specs=pl.BlockSpec((NB, DIM), lambda i: (i, 0)),
        out_shape=jax.ShapeDtypeStruct((N, DIM), jnp.float32),
    )(x, w_root, b2d, acc)


def _copy_out_clipped(acc, out_hbm, c, s):
    @pl.when(s < NS - 1)
    def _():
        pltpu.sync_copy(acc.at[pl.ds(s * ROWS_PER_SUB, ROWS_PER_SUB)],
                        out_hbm.at[c, pl.ds(s * ROWS_PER_SUB, ROWS_PER_SUB)])

    @pl.when(s == NS - 1)
    def _():
        pltpu.sync_copy(
            acc.at[pl.ds((NS - 1) * ROWS_PER_SUB, N - (NS - 1) * ROWS_PER_SUB)],
            out_hbm.at[c, pl.ds((NS - 1) * ROWS_PER_SUB,
                                N - (NS - 1) * ROWS_PER_SUB)])


@functools.partial(
    pl.kernel,
    out_type=jax.ShapeDtypeStruct((NC, N, DIM), jnp.float32),
    mesh=_mesh,
    scratch_types=[
        pltpu.VMEM((1, G), jnp.int32),
        pltpu.VMEM((1, G), jnp.int32),
        pltpu.VMEM((G, DIM), jnp.float32),
        pltpu.VMEM_SHARED((ACC_ROWS, DIM), jnp.float32),
    ],
)
def _count_pass(et_hbm, dst_hbm, oh_hbm, zeros_hbm, out_hbm,
                t_v, dst_v, oh_v, acc):
    c = lax.axis_index("c")
    s = lax.axis_index("s")
    wid = s * NC + c
    pltpu.sync_copy(zeros_hbm, acc.at[pl.ds(s * ROWS_PER_SUB, ROWS_PER_SUB)])
    plsc.subcore_barrier()

    @pl.loop(0, KMAX)
    def _(k):
        widx = wid + NW * k

        @pl.when(widx < NWIN)
        def _():
            base = widx * G
            pltpu.sync_copy(et_hbm.at[pl.ds(base, G)], t_v.at[0])
            pltpu.sync_copy(dst_hbm.at[pl.ds(base, G)], dst_v.at[0])
            pltpu.sync_copy(oh_hbm.at[t_v.at[0]], oh_v)
            pltpu.sync_copy(oh_v, acc.at[dst_v.at[0]], add=True)

    plsc.subcore_barrier()
    _copy_out_clipped(acc, out_hbm, c, s)


@functools.partial(
    pl.kernel,
    out_type=jax.ShapeDtypeStruct((E,), jnp.float32),
    mesh=_mesh,
    scratch_types=[
        pltpu.VMEM((1, G), jnp.int32),
        pltpu.VMEM((1, G), jnp.float32),
        pltpu.VMEM((NR,), jnp.float32),
    ],
    compiler_params=_sc_params,
)
def _scale_pass(sidx_hbm, inv_hbm, scl_hbm, sidx_v, scl_v, inv_v):
    c = lax.axis_index("c")
    s = lax.axis_index("s")
    wid = s * NC + c
    pltpu.sync_copy(inv_hbm, inv_v)

    @pl.loop(0, KMAX)
    def _(k):
        widx = wid + NW * k

        @pl.when(widx < NWIN)
        def _():
            base = widx * G
            pltpu.sync_copy(sidx_hbm.at[pl.ds(base, G)], sidx_v.at[0])

            for jj in range(G // 16):
                idxv = sidx_v[0, pl.ds(jj * 16, 16)]
                scl_v[0, pl.ds(jj * 16, 16)] = plsc.load_gather(inv_v, [idxv])

            pltpu.sync_copy(scl_v.at[0], scl_hbm.at[pl.ds(base, G)])


@functools.partial(
    pl.kernel,
    out_type=jax.ShapeDtypeStruct((NC, N, DIM), jnp.float32),
    mesh=_mesh,
    scratch_types=[
        pltpu.VMEM((1, G), jnp.int32),
        pltpu.VMEM((1, G), jnp.int32),
        pltpu.VMEM((1, G), jnp.float32),
        pltpu.VMEM((G, DIM), jnp.float32),
        pltpu.VMEM_SHARED((ACC_ROWS, DIM), jnp.float32),
    ],
    compiler_params=_sc_params,
)
def _edge_pass(gidx_hbm, dst_hbm, scl_hbm, h_hbm, zeros_hbm,
               out_hbm, gidx_v, dst_v, scl_v, rows_v, acc):
    c = lax.axis_index("c")
    s = lax.axis_index("s")
    wid = s * NC + c
    pltpu.sync_copy(zeros_hbm,
                    acc.at[pl.ds(s * ROWS_PER_SUB, ROWS_PER_SUB)])
    plsc.subcore_barrier()

    @pl.loop(0, KMAX)
    def _(k):
        widx = wid + NW * k

        @pl.when(widx < NWIN)
        def _():
            base = widx * G
            pltpu.sync_copy(gidx_hbm.at[pl.ds(base, G)], gidx_v.at[0])
            pltpu.sync_copy(scl_hbm.at[pl.ds(base, G)], scl_v.at[0])
            pltpu.sync_copy(dst_hbm.at[pl.ds(base, G)], dst_v.at[0])
            pltpu.sync_copy(h_hbm.at[gidx_v.at[0]], rows_v)

            @pl.loop(0, G)
            def _(j):
                jfull = jnp.full((16,), j, jnp.int32)
                sval = plsc.load_gather(
                    scl_v, [jnp.zeros((16,), jnp.int32), jfull])
                for cc in range(DIM // 16):
                    rows_v[j, pl.ds(cc * 16, 16)] = (
                        rows_v[j, pl.ds(cc * 16, 16)] * sval)

            pltpu.sync_copy(rows_v, acc.at[dst_v.at[0]], add=True)

    plsc.subcore_barrier()
    _copy_out_clipped(acc, out_hbm, c, s)


def kernel(edge_index, edge_type, emb, W_rel1, W_root1, b1, W_rel2, W_root2, b2):
    src = edge_index[0].astype(jnp.int32)
    dst = edge_index[1].astype(jnp.int32)
    et = edge_type.astype(jnp.int32)
    gidx = et * N + src
    sidx = dst * R + et

    zeros_nd = jnp.zeros((ROWS_PER_SUB, DIM), jnp.float32)
    oh = (jnp.arange(DIM, dtype=jnp.int32)[None, :] // 16
          == jnp.arange(R, dtype=jnp.int32)[:, None]).astype(jnp.float32)

    cnt2 = _count_pass(et, dst, oh, zeros_nd)
    inv_nd = _inv_counts(cnt2)
    inv_c = inv_nd.reshape(N, R, 16)[:, :, 0].reshape(NR)
    scl = _scale_pass(sidx, inv_c)

    h1 = _h_all(emb, W_rel1).reshape(R * N, DIM)
    acc1 = _edge_pass(gidx, dst, scl, h1, zeros_nd)
    x1 = _combine(emb, W_root1, b1.reshape(1, DIM), acc1)

    h2 = _h_all(x1, W_rel2).reshape(R * N, DIM)
    acc2 = _edge_pass(gidx, dst, scl, h2, zeros_nd)
    x2 = _combine(x1, W_root2, b2.reshape(1, DIM), acc2)
    return x2


# speedup vs baseline: 14.6284x; 2.1272x over previous
import dataclasses
import functools

import jax
import jax.numpy as jnp
from jax import lax
from jax.experimental import pallas as pl
from jax.experimental.pallas import tpu as pltpu
from jax.experimental.pallas import tpu_sc as plsc

N = 10000
R = 8
DIM = 128
E = 320000
G = 128
NWIN = E // G
NC = 2
NS = 16
NW = NC * NS
KMAX = (NWIN + NW - 1) // NW
NR = N * R
ROWS_PER_SUB = 640
ACC_ROWS = NS * ROWS_PER_SUB
CROWS_PER_SUB = NR // NS
NB = 2000

_mesh = plsc.VectorSubcoreMesh(core_axis_name="c", subcore_axis_name="s")

_sc_params = pltpu.CompilerParams()
if "needs_layout_passes" in pltpu.CompilerParams.__dataclass_fields__:
    _sc_params = dataclasses.replace(_sc_params, needs_layout_passes=False)


def _h_body(x_ref, w_ref, h_ref):
    h_ref[0] = jnp.dot(x_ref[...], w_ref[0], preferred_element_type=jnp.float32)


def _h_all(x, w_rel):
    return pl.pallas_call(
        _h_body,
        grid=(N // NB, R),
        in_specs=[
            pl.BlockSpec((NB, DIM), lambda i, r: (i, 0)),
            pl.BlockSpec((1, DIM, DIM), lambda i, r: (r, 0, 0)),
        ],
        out_specs=pl.BlockSpec((1, NB, DIM), lambda i, r: (r, i, 0)),
        out_shape=jax.ShapeDtypeStruct((R, N, DIM), jnp.float32),
    )(x, w_rel)


def _inv_body(c_ref, o_ref):
    o_ref[...] = 1.0 / jnp.maximum(c_ref[0] + c_ref[1], 1.0)


def _inv_counts(cnt2):
    return pl.pallas_call(
        _inv_body,
        grid=(N // NB,),
        in_specs=[pl.BlockSpec((2, NB, DIM), lambda i: (0, i, 0))],
        out_specs=pl.BlockSpec((NB, DIM), lambda i: (i, 0)),
        out_shape=jax.ShapeDtypeStruct((N, DIM), jnp.float32),
    )(cnt2)


def _combine_body(x_ref, w_ref, b_ref, a_ref, o_ref):
    y = jnp.dot(x_ref[...], w_ref[...], preferred_element_type=jnp.float32)
    o_ref[...] = jnp.maximum(y + b_ref[...] + a_ref[0] + a_ref[1], 0.0)


def _combine(x, w_root, b2d, acc):
    return pl.pallas_call(
        _combine_body,
        grid=(N // NB,),
        in_specs=[
            pl.BlockSpec((NB, DIM), lambda i: (i, 0)),
            pl.BlockSpec((DIM, DIM), lambda i: (0, 0)),
            pl.BlockSpec((1, DIM), lambda i: (0, 0)),
            pl.BlockSpec((2, NB, DIM), lambda i: (0, i, 0)),
        ],
        out_specs=pl.BlockSpec((NB, DIM), lambda i: (i, 0)),
        out_shape=jax.ShapeDtypeStruct((N, DIM), jnp.float32),
    )(x, w_root, b2d, acc)


def _copy_out_clipped(acc, out_hbm, c, s):
    @pl.when(s < NS - 1)
    def _():
        pltpu.sync_copy(acc.at[pl.ds(s * ROWS_PER_SUB, ROWS_PER_SUB)],
                        out_hbm.at[c, pl.ds(s * ROWS_PER_SUB, ROWS_PER_SUB)])

    @pl.when(s == NS - 1)
    def _():
        pltpu.sync_copy(
            acc.at[pl.ds((NS - 1) * ROWS_PER_SUB, N - (NS - 1) * ROWS_PER_SUB)],
            out_hbm.at[c, pl.ds((NS - 1) * ROWS_PER_SUB,
                                N - (NS - 1) * ROWS_PER_SUB)])


@functools.partial(
    pl.kernel,
    out_type=jax.ShapeDtypeStruct((NC, N, DIM), jnp.float32),
    mesh=_mesh,
    scratch_types=[
        pltpu.VMEM((1, G), jnp.int32),
        pltpu.VMEM((1, G), jnp.int32),
        pltpu.VMEM((G, DIM), jnp.float32),
        pltpu.VMEM_SHARED((ACC_ROWS, DIM), jnp.float32),
    ],
    compiler_params=_sc_params,
)
def _count_pass(et_hbm, dst_hbm, zeros_hbm, out_hbm, t_v, dst_v, oh_v, acc):
    c = lax.axis_index("c")
    s = lax.axis_index("s")
    wid = s * NC + c
    pltpu.sync_copy(zeros_hbm, acc.at[pl.ds(s * ROWS_PER_SUB, ROWS_PER_SUB)])
    plsc.subcore_barrier()

    @pl.loop(0, KMAX)
    def _(k):
        widx = wid + NW * k

        @pl.when(widx < NWIN)
        def _():
            base = widx * G
            pltpu.sync_copy(et_hbm.at[pl.ds(base, G)], t_v.at[0])
            pltpu.sync_copy(dst_hbm.at[pl.ds(base, G)], dst_v.at[0])

            @pl.loop(0, G)
            def _(j):
                jfull = jnp.full((16,), j, jnp.int32)
                t_splat = plsc.load_gather(
                    t_v, [jnp.zeros((16,), jnp.int32), jfull])
                for cc in range(DIM // 16):
                    oh_v[j, pl.ds(cc * 16, 16)] = jnp.where(
                        t_splat == cc, 1.0, 0.0).astype(jnp.float32)

            pltpu.sync_copy(oh_v, acc.at[dst_v.at[0]], add=True)

    plsc.subcore_barrier()
    _copy_out_clipped(acc, out_hbm, c, s)


@functools.partial(
    pl.kernel,
    out_type=jax.ShapeDtypeStruct((E,), jnp.float32),
    mesh=_mesh,
    scratch_types=[
        pltpu.VMEM((1, G), jnp.int32),
        pltpu.VMEM((1, G), jnp.float32),
        pltpu.VMEM((NR,), jnp.float32),
    ],
    compiler_params=_sc_params,
)
def _scale_pass(sidx_hbm, inv_hbm, scl_hbm, sidx_v, scl_v, inv_v):
    c = lax.axis_index("c")
    s = lax.axis_index("s")
    wid = s * NC + c
    pltpu.sync_copy(inv_hbm, inv_v)

    @pl.loop(0, KMAX)
    def _(k):
        widx = wid + NW * k

        @pl.when(widx < NWIN)
        def _():
            base = widx * G
            pltpu.sync_copy(sidx_hbm.at[pl.ds(base, G)], sidx_v.at[0])

            for jj in range(G // 16):
                idxv = sidx_v[0, pl.ds(jj * 16, 16)]
                scl_v[0, pl.ds(jj * 16, 16)] = plsc.load_gather(inv_v, [idxv])

            pltpu.sync_copy(scl_v.at[0], scl_hbm.at[pl.ds(base, G)])


@functools.partial(
    pl.kernel,
    out_type=jax.ShapeDtypeStruct((NC, N, DIM), jnp.float32),
    mesh=_mesh,
    scratch_types=[
        pltpu.VMEM((1, G), jnp.int32),
        pltpu.VMEM((1, G), jnp.int32),
        pltpu.VMEM((1, G), jnp.float32),
        pltpu.VMEM((G, DIM), jnp.float32),
        pltpu.VMEM_SHARED((ACC_ROWS, DIM), jnp.float32),
    ],
    compiler_params=_sc_params,
)
def _edge_pass(gidx_hbm, dst_hbm, scl_hbm, h_hbm, zeros_hbm,
               out_hbm, gidx_v, dst_v, scl_v, rows_v, acc):
    c = lax.axis_index("c")
    s = lax.axis_index("s")
    wid = s * NC + c
    pltpu.sync_copy(zeros_hbm,
                    acc.at[pl.ds(s * ROWS_PER_SUB, ROWS_PER_SUB)])
    plsc.subcore_barrier()

    @pl.loop(0, KMAX)
    def _(k):
        widx = wid + NW * k

        @pl.when(widx < NWIN)
        def _():
            base = widx * G
            pltpu.sync_copy(gidx_hbm.at[pl.ds(base, G)], gidx_v.at[0])
            pltpu.sync_copy(scl_hbm.at[pl.ds(base, G)], scl_v.at[0])
            pltpu.sync_copy(dst_hbm.at[pl.ds(base, G)], dst_v.at[0])
            pltpu.sync_copy(h_hbm.at[gidx_v.at[0]], rows_v)

            @pl.loop(0, G)
            def _(j):
                jfull = jnp.full((16,), j, jnp.int32)
                sval = plsc.load_gather(
                    scl_v, [jnp.zeros((16,), jnp.int32), jfull])
                for cc in range(DIM // 16):
                    rows_v[j, pl.ds(cc * 16, 16)] = (
                        rows_v[j, pl.ds(cc * 16, 16)] * sval)

            pltpu.sync_copy(rows_v, acc.at[dst_v.at[0]], add=True)

    plsc.subcore_barrier()
    _copy_out_clipped(acc, out_hbm, c, s)


def kernel(edge_index, edge_type, emb, W_rel1, W_root1, b1, W_rel2, W_root2, b2):
    src = edge_index[0].astype(jnp.int32)
    dst = edge_index[1].astype(jnp.int32)
    et = edge_type.astype(jnp.int32)
    gidx = et * N + src
    sidx = dst * R + et

    zeros_nd = jnp.zeros((ROWS_PER_SUB, DIM), jnp.float32)

    cnt2 = _count_pass(et, dst, zeros_nd)
    inv_nd = _inv_counts(cnt2)
    inv_c = inv_nd.reshape(N, R, 16)[:, :, 0].reshape(NR)
    scl = _scale_pass(sidx, inv_c)

    h1 = _h_all(emb, W_rel1).reshape(R * N, DIM)
    acc1 = _edge_pass(gidx, dst, scl, h1, zeros_nd)
    x1 = _combine(emb, W_root1, b1.reshape(1, DIM), acc1)

    h2 = _h_all(x1, W_rel2).reshape(R * N, DIM)
    acc2 = _edge_pass(gidx, dst, scl, h2, zeros_nd)
    x2 = _combine(x1, W_root2, b2.reshape(1, DIM), acc2)
    return x2


# speedup vs baseline: 17.5643x; 1.2007x over previous
import dataclasses
import functools

import jax
import jax.numpy as jnp
from jax import lax
from jax.experimental import pallas as pl
from jax.experimental.pallas import tpu as pltpu
from jax.experimental.pallas import tpu_sc as plsc

N = 10000
R = 8
DIM = 128
E = 320000
G = 128
NWIN = E // G
NC = 2
NS = 16
NW = NC * NS
KMAX = (NWIN + NW - 1) // NW
NR = N * R
ROWS_PER_SUB = 640
ACC_ROWS = NS * ROWS_PER_SUB
CROWS_PER_SUB = NR // NS
NB = 2000

_mesh = plsc.VectorSubcoreMesh(core_axis_name="c", subcore_axis_name="s")

_sc_params = pltpu.CompilerParams()
if "needs_layout_passes" in pltpu.CompilerParams.__dataclass_fields__:
    _sc_params = dataclasses.replace(_sc_params, needs_layout_passes=False)


def _h_body(x_ref, w_ref, h_ref):
    h_ref[0] = jnp.dot(x_ref[...], w_ref[0], preferred_element_type=jnp.float32)


def _h_all(x, w_rel):
    return pl.pallas_call(
        _h_body,
        grid=(N // NB, R),
        in_specs=[
            pl.BlockSpec((NB, DIM), lambda i, r: (i, 0)),
            pl.BlockSpec((1, DIM, DIM), lambda i, r: (r, 0, 0)),
        ],
        out_specs=pl.BlockSpec((1, NB, DIM), lambda i, r: (r, i, 0)),
        out_shape=jax.ShapeDtypeStruct((R, N, DIM), jnp.float32),
    )(x, w_rel)


def _inv_body(c_ref, o_ref):
    o_ref[...] = 1.0 / jnp.maximum(c_ref[0] + c_ref[1], 1.0)


def _inv_counts(cnt2):
    return pl.pallas_call(
        _inv_body,
        grid=(N // NB,),
        in_specs=[pl.BlockSpec((2, NB, DIM), lambda i: (0, i, 0))],
        out_specs=pl.BlockSpec((NB, DIM), lambda i: (i, 0)),
        out_shape=jax.ShapeDtypeStruct((N, DIM), jnp.float32),
    )(cnt2)


def _combine_body(x_ref, w_ref, b_ref, a_ref, o_ref):
    y = jnp.dot(x_ref[...], w_ref[...], preferred_element_type=jnp.float32)
    o_ref[...] = jnp.maximum(y + b_ref[...] + a_ref[0] + a_ref[1], 0.0)


def _combine(x, w_root, b2d, acc):
    return pl.pallas_call(
        _combine_body,
        grid=(N // NB,),
        in_specs=[
            pl.BlockSpec((NB, DIM), lambda i: (i, 0)),
            pl.BlockSpec((DIM, DIM), lambda i: (0, 0)),
            pl.BlockSpec((1, DIM), lambda i: (0, 0)),
            pl.BlockSpec((2, NB, DIM), lambda i: (0, i, 0)),
        ],
        out_specs=pl.BlockSpec((NB, DIM), lambda i: (i, 0)),
        out_shape=jax.ShapeDtypeStruct((N, DIM), jnp.float32),
    )(x, w_root, b2d, acc)


def _copy_out_clipped(acc, out_hbm, c, s):
    @pl.when(s < NS - 1)
    def _():
        pltpu.sync_copy(acc.at[pl.ds(s * ROWS_PER_SUB, ROWS_PER_SUB)],
                        out_hbm.at[c, pl.ds(s * ROWS_PER_SUB, ROWS_PER_SUB)])

    @pl.when(s == NS - 1)
    def _():
        pltpu.sync_copy(
            acc.at[pl.ds((NS - 1) * ROWS_PER_SUB, N - (NS - 1) * ROWS_PER_SUB)],
            out_hbm.at[c, pl.ds((NS - 1) * ROWS_PER_SUB,
                                N - (NS - 1) * ROWS_PER_SUB)])


@functools.partial(
    pl.kernel,
    out_type=jax.ShapeDtypeStruct((NC, N, DIM), jnp.float32),
    mesh=_mesh,
    scratch_types=[
        pltpu.VMEM((1, G), jnp.int32),
        pltpu.VMEM((1, G), jnp.int32),
        pltpu.VMEM((G, DIM), jnp.float32),
        pltpu.VMEM_SHARED((ACC_ROWS, DIM), jnp.float32),
    ],
    compiler_params=_sc_params,
)
def _count_pass(et_hbm, dst_hbm, zeros_hbm, out_hbm, t_v, dst_v, oh_v, acc):
    c = lax.axis_index("c")
    s = lax.axis_index("s")
    wid = s * NC + c
    pltpu.sync_copy(zeros_hbm, acc.at[pl.ds(s * ROWS_PER_SUB, ROWS_PER_SUB)])
    plsc.subcore_barrier()

    @pl.loop(0, KMAX)
    def _(k):
        widx = wid + NW * k

        @pl.when(widx < NWIN)
        def _():
            base = widx * G
            pltpu.sync_copy(et_hbm.at[pl.ds(base, G)], t_v.at[0])
            pltpu.sync_copy(dst_hbm.at[pl.ds(base, G)], dst_v.at[0])

            @pl.loop(0, G)
            def _(j):
                jfull = jnp.full((16,), j, jnp.int32)
                t_splat = plsc.load_gather(
                    t_v, [jnp.zeros((16,), jnp.int32), jfull])
                for cc in range(DIM // 16):
                    oh_v[j, pl.ds(cc * 16, 16)] = jnp.where(
                        t_splat == cc, 1.0, 0.0).astype(jnp.float32)

            pltpu.sync_copy(oh_v, acc.at[dst_v.at[0]], add=True)

    plsc.subcore_barrier()
    _copy_out_clipped(acc, out_hbm, c, s)


@functools.partial(
    pl.kernel,
    out_type=jax.ShapeDtypeStruct((E,), jnp.float32),
    mesh=_mesh,
    scratch_types=[
        pltpu.VMEM((1, G), jnp.int32),
        pltpu.VMEM((1, G), jnp.float32),
        pltpu.VMEM((NR,), jnp.float32),
    ],
    compiler_params=_sc_params,
)
def _scale_pass(sidx_hbm, inv_hbm, scl_hbm, sidx_v, scl_v, inv_v):
    c = lax.axis_index("c")
    s = lax.axis_index("s")
    wid = s * NC + c
    pltpu.sync_copy(inv_hbm, inv_v)

    @pl.loop(0, KMAX)
    def _(k):
        widx = wid + NW * k

        @pl.when(widx < NWIN)
        def _():
            base = widx * G
            pltpu.sync_copy(sidx_hbm.at[pl.ds(base, G)], sidx_v.at[0])

            for jj in range(G // 16):
                idxv = sidx_v[0, pl.ds(jj * 16, 16)]
                scl_v[0, pl.ds(jj * 16, 16)] = plsc.load_gather(inv_v, [idxv])

            pltpu.sync_copy(scl_v.at[0], scl_hbm.at[pl.ds(base, G)])


@functools.partial(
    pl.kernel,
    out_type=jax.ShapeDtypeStruct((NC, N, DIM), jnp.float32),
    mesh=_mesh,
    scratch_types=[
        pltpu.VMEM((1, G), jnp.int32),
        pltpu.VMEM((1, G), jnp.int32),
        pltpu.VMEM((1, G), jnp.float32),
        pltpu.VMEM((G, DIM), jnp.float32),
        pltpu.VMEM((1, G), jnp.int32),
        pltpu.VMEM((1, G), jnp.int32),
        pltpu.VMEM((1, G), jnp.float32),
        pltpu.VMEM((G, DIM), jnp.float32),
        pltpu.SemaphoreType.DMA,
        pltpu.SemaphoreType.DMA,
        pltpu.VMEM_SHARED((ACC_ROWS, DIM), jnp.float32),
    ],
    compiler_params=_sc_params,
)
def _edge_pass(gidx_hbm, dst_hbm, scl_hbm, h_hbm, zeros_hbm, out_hbm,
               gidx_v0, dst_v0, scl_v0, rows_v0,
               gidx_v1, dst_v1, scl_v1, rows_v1,
               sem0, sem1, acc):
    c = lax.axis_index("c")
    s = lax.axis_index("s")
    wid = s * NC + c
    pltpu.sync_copy(zeros_hbm,
                    acc.at[pl.ds(s * ROWS_PER_SUB, ROWS_PER_SUB)])
    plsc.subcore_barrier()

    def issue(widx, gidx_v, dst_v, scl_v, rows_v, sem):
        base = widx * G
        pltpu.sync_copy(gidx_hbm.at[pl.ds(base, G)], gidx_v.at[0])
        pltpu.sync_copy(scl_hbm.at[pl.ds(base, G)], scl_v.at[0])
        pltpu.sync_copy(dst_hbm.at[pl.ds(base, G)], dst_v.at[0])
        pltpu.async_copy(h_hbm.at[gidx_v.at[0]], rows_v, sem)

    def finish(gidx_v, dst_v, scl_v, rows_v, sem):
        pltpu.make_async_copy(h_hbm.at[gidx_v.at[0]], rows_v, sem).wait()

        @pl.loop(0, G)
        def _(j):
            jfull = jnp.full((16,), j, jnp.int32)
            sval = plsc.load_gather(
                scl_v, [jnp.zeros((16,), jnp.int32), jfull])
            for cc in range(DIM // 16):
                rows_v[j, pl.ds(cc * 16, 16)] = (
                    rows_v[j, pl.ds(cc * 16, 16)] * sval)

        pltpu.sync_copy(rows_v, acc.at[dst_v.at[0]], add=True)

    @pl.loop(0, KMAX + 1, step=2)
    def _(k):
        w0 = wid + NW * k
        w1 = wid + NW * (k + 1)

        @pl.when(w0 < NWIN)
        def _():
            issue(w0, gidx_v0, dst_v0, scl_v0, rows_v0, sem0)

        @pl.when(w1 < NWIN)
        def _():
            issue(w1, gidx_v1, dst_v1, scl_v1, rows_v1, sem1)

        @pl.when(w0 < NWIN)
        def _():
            finish(gidx_v0, dst_v0, scl_v0, rows_v0, sem0)

        @pl.when(w1 < NWIN)
        def _():
            finish(gidx_v1, dst_v1, scl_v1, rows_v1, sem1)

    plsc.subcore_barrier()
    _copy_out_clipped(acc, out_hbm, c, s)


def kernel(edge_index, edge_type, emb, W_rel1, W_root1, b1, W_rel2, W_root2, b2):
    src = edge_index[0].astype(jnp.int32)
    dst = edge_index[1].astype(jnp.int32)
    et = edge_type.astype(jnp.int32)
    gidx = et * N + src
    sidx = dst * R + et

    zeros_nd = jnp.zeros((ROWS_PER_SUB, DIM), jnp.float32)

    cnt2 = _count_pass(et, dst, zeros_nd)
    inv_nd = _inv_counts(cnt2)
    inv_c = inv_nd.reshape(N, R, 16)[:, :, 0].reshape(NR)
    scl = _scale_pass(sidx, inv_c)

    h1 = _h_all(emb, W_rel1).reshape(R * N, DIM)
    acc1 = _edge_pass(gidx, dst, scl, h1, zeros_nd)
    x1 = _combine(emb, W_root1, b1.reshape(1, DIM), acc1)

    h2 = _h_all(x1, W_rel2).reshape(R * N, DIM)
    acc2 = _edge_pass(gidx, dst, scl, h2, zeros_nd)
    x2 = _combine(x1, W_root2, b2.reshape(1, DIM), acc2)
    return x2


# speedup vs baseline: 22.6022x; 1.2868x over previous
import dataclasses
import functools

import jax
import jax.numpy as jnp
from jax import lax
from jax.experimental import pallas as pl
from jax.experimental.pallas import tpu as pltpu
from jax.experimental.pallas import tpu_sc as plsc

N = 10000
R = 8
DIM = 128
E = 320000
G = 128
NWIN = E // G
NC = 2
NS = 16
NW = NC * NS
KMAX = (NWIN + NW - 1) // NW
NR = N * R
ROWS_PER_SUB = 640
ACC_ROWS = NS * ROWS_PER_SUB
CROWS_PER_SUB = NR // NS
NB = 2000

_mesh = plsc.VectorSubcoreMesh(core_axis_name="c", subcore_axis_name="s")

_sc_params = pltpu.CompilerParams()
if "needs_layout_passes" in pltpu.CompilerParams.__dataclass_fields__:
    _sc_params = dataclasses.replace(_sc_params, needs_layout_passes=False)


def _h_body(x_ref, w_ref, h_ref):
    h_ref[0] = jnp.dot(x_ref[...], w_ref[0], preferred_element_type=jnp.float32)


def _h_all(x, w_rel):
    return pl.pallas_call(
        _h_body,
        grid=(N // NB, R),
        in_specs=[
            pl.BlockSpec((NB, DIM), lambda i, r: (i, 0)),
            pl.BlockSpec((1, DIM, DIM), lambda i, r: (r, 0, 0)),
        ],
        out_specs=pl.BlockSpec((1, NB, DIM), lambda i, r: (r, i, 0)),
        out_shape=jax.ShapeDtypeStruct((R, N, DIM), jnp.float32),
    )(x, w_rel)


def _inv_body(c_ref, o_ref):
    o_ref[...] = 1.0 / jnp.maximum(c_ref[0] + c_ref[1], 1.0)


def _inv_counts(cnt2):
    return pl.pallas_call(
        _inv_body,
        grid=(N // NB,),
        in_specs=[pl.BlockSpec((2, NB, DIM), lambda i: (0, i, 0))],
        out_specs=pl.BlockSpec((NB, DIM), lambda i: (i, 0)),
        out_shape=jax.ShapeDtypeStruct((N, DIM), jnp.float32),
    )(cnt2)


def _combine_body(x_ref, w_ref, b_ref, a_ref, o_ref):
    y = jnp.dot(x_ref[...], w_ref[...], preferred_element_type=jnp.float32)
    o_ref[...] = jnp.maximum(y + b_ref[...] + a_ref[0] + a_ref[1], 0.0)


def _combine(x, w_root, b2d, acc):
    return pl.pallas_call(
        _combine_body,
        grid=(N // NB,),
        in_specs=[
            pl.BlockSpec((NB, DIM), lambda i: (i, 0)),
            pl.BlockSpec((DIM, DIM), lambda i: (0, 0)),
            pl.BlockSpec((1, DIM), lambda i: (0, 0)),
            pl.BlockSpec((2, NB, DIM), lambda i: (0, i, 0)),
        ],
        out_specs=pl.BlockSpec((NB, DIM), lambda i: (i, 0)),
        out_shape=jax.ShapeDtypeStruct((N, DIM), jnp.float32),
    )(x, w_root, b2d, acc)


def _copy_out_clipped(acc, out_hbm, c, s):
    @pl.when(s < NS - 1)
    def _():
        pltpu.sync_copy(acc.at[pl.ds(s * ROWS_PER_SUB, ROWS_PER_SUB)],
                        out_hbm.at[c, pl.ds(s * ROWS_PER_SUB, ROWS_PER_SUB)])

    @pl.when(s == NS - 1)
    def _():
        pltpu.sync_copy(
            acc.at[pl.ds((NS - 1) * ROWS_PER_SUB, N - (NS - 1) * ROWS_PER_SUB)],
            out_hbm.at[c, pl.ds((NS - 1) * ROWS_PER_SUB,
                                N - (NS - 1) * ROWS_PER_SUB)])


@functools.partial(
    pl.kernel,
    out_type=jax.ShapeDtypeStruct((NC, N, DIM), jnp.float32),
    mesh=_mesh,
    scratch_types=[
        pltpu.VMEM((1, G), jnp.int32),
        pltpu.VMEM((1, G), jnp.int32),
        pltpu.VMEM((G, DIM), jnp.float32),
        pltpu.VMEM_SHARED((ACC_ROWS, DIM), jnp.float32),
    ],
    compiler_params=_sc_params,
)
def _count_pass(et_hbm, dst_hbm, zeros_hbm, out_hbm, t_v, dst_v, oh_v, acc):
    c = lax.axis_index("c")
    s = lax.axis_index("s")
    wid = s * NC + c
    pltpu.sync_copy(zeros_hbm, acc.at[pl.ds(s * ROWS_PER_SUB, ROWS_PER_SUB)])
    plsc.subcore_barrier()

    @pl.loop(0, KMAX)
    def _(k):
        widx = wid + NW * k

        @pl.when(widx < NWIN)
        def _():
            base = widx * G
            pltpu.sync_copy(et_hbm.at[pl.ds(base, G)], t_v.at[0])
            pltpu.sync_copy(dst_hbm.at[pl.ds(base, G)], dst_v.at[0])

            @pl.loop(0, G)
            def _(j):
                jfull = jnp.full((16,), j, jnp.int32)
                t_splat = plsc.load_gather(
                    t_v, [jnp.zeros((16,), jnp.int32), jfull])
                for cc in range(DIM // 16):
                    oh_v[j, pl.ds(cc * 16, 16)] = jnp.where(
                        t_splat == cc, 1.0, 0.0).astype(jnp.float32)

            pltpu.sync_copy(oh_v, acc.at[dst_v.at[0]], add=True)

    plsc.subcore_barrier()
    _copy_out_clipped(acc, out_hbm, c, s)


@functools.partial(
    pl.kernel,
    out_type=jax.ShapeDtypeStruct((E,), jnp.float32),
    mesh=_mesh,
    scratch_types=[
        pltpu.VMEM((1, G), jnp.int32),
        pltpu.VMEM((1, G), jnp.float32),
        pltpu.VMEM((NR,), jnp.float32),
    ],
    compiler_params=_sc_params,
)
def _scale_pass(sidx_hbm, inv_hbm, scl_hbm, sidx_v, scl_v, inv_v):
    c = lax.axis_index("c")
    s = lax.axis_index("s")
    wid = s * NC + c
    pltpu.sync_copy(inv_hbm, inv_v)

    @pl.loop(0, KMAX)
    def _(k):
        widx = wid + NW * k

        @pl.when(widx < NWIN)
        def _():
            base = widx * G
            pltpu.sync_copy(sidx_hbm.at[pl.ds(base, G)], sidx_v.at[0])

            for jj in range(G // 16):
                idxv = sidx_v[0, pl.ds(jj * 16, 16)]
                scl_v[0, pl.ds(jj * 16, 16)] = plsc.load_gather(inv_v, [idxv])

            pltpu.sync_copy(scl_v.at[0], scl_hbm.at[pl.ds(base, G)])


@functools.partial(
    pl.kernel,
    out_type=jax.ShapeDtypeStruct((NC, N, DIM), jnp.float32),
    mesh=_mesh,
    scratch_types=[
        pltpu.VMEM((8, G), jnp.int32),
        pltpu.VMEM((8, G), jnp.int32),
        pltpu.VMEM((G, DIM), jnp.float32),
        pltpu.VMEM((G, DIM), jnp.float32),
        pltpu.VMEM((1, G), jnp.int32),
        pltpu.VMEM((1, G), jnp.int32),
        pltpu.SemaphoreType.DMA,
        pltpu.SemaphoreType.DMA,
        pltpu.SemaphoreType.DMA,
        pltpu.SemaphoreType.DMA,
        pltpu.SemaphoreType.DMA,
        pltpu.SemaphoreType.DMA,
        pltpu.VMEM_SHARED((ACC_ROWS, DIM), jnp.float32),
    ],
    compiler_params=_sc_params,
)
def _edge_pass(pk_hbm, h_hbm, zeros_hbm, out_hbm,
               pk0, pk1, rows0, rows1, dstb0, dstb1,
               sp0, sp1, sr0, sr1, ss0, ss1, acc):
    c = lax.axis_index("c")
    s = lax.axis_index("s")
    wid = s * NC + c
    pltpu.sync_copy(zeros_hbm,
                    acc.at[pl.ds(s * ROWS_PER_SUB, ROWS_PER_SUB)])
    plsc.subcore_barrier()

    def valid(n):
        return (wid + NW * n) < NWIN

    def issue_pk(n, pk, sp):
        pltpu.async_copy(pk_hbm.at[wid + NW * n], pk, sp)

    def wait_pk(pk, sp):
        pltpu.make_async_copy(pk_hbm.at[0], pk, sp).wait()

    def issue_gather(pk, rows, sr):
        pltpu.async_copy(h_hbm.at[pk.at[0]], rows, sr)

    def wait_gather(pk, rows, sr):
        pltpu.make_async_copy(h_hbm.at[pk.at[0]], rows, sr).wait()

    def compute(pk, rows, dstb):
        @pl.loop(0, G)
        def _(j):
            jfull = jnp.full((16,), j, jnp.int32)
            sbits = plsc.load_gather(
                pk, [jnp.full((16,), 1, jnp.int32), jfull])
            sval = plsc.bitcast(sbits, jnp.float32)
            for cc in range(DIM // 16):
                rows[j, pl.ds(cc * 16, 16)] = (
                    rows[j, pl.ds(cc * 16, 16)] * sval)

        for cc in range(G // 16):
            dstb[0, pl.ds(cc * 16, 16)] = pk[2, pl.ds(cc * 16, 16)]

    def issue_scatter(rows, dstb, ss):
        pltpu.async_copy(rows, acc.at[dstb.at[0]], ss, add=True)

    def wait_scatter(rows, dstb, ss):
        pltpu.make_async_copy(rows, acc.at[dstb.at[0]], ss).wait()

    @pl.when(valid(0))
    def _():
        issue_pk(0, pk0, sp0)

    @pl.when(valid(1))
    def _():
        issue_pk(1, pk1, sp1)

    @pl.when(valid(0))
    def _():
        wait_pk(pk0, sp0)
        issue_gather(pk0, rows0, sr0)

    @pl.loop(0, KMAX + 1, step=2)
    def _(k):
        @pl.when(valid(k + 1))
        def _():
            wait_pk(pk1, sp1)

            @pl.when(k >= 1)
            def _():
                wait_scatter(rows1, dstb1, ss1)

            issue_gather(pk1, rows1, sr1)

        @pl.when(valid(k))
        def _():
            wait_gather(pk0, rows0, sr0)
            compute(pk0, rows0, dstb0)
            issue_scatter(rows0, dstb0, ss0)

        @pl.when(valid(k + 2))
        def _():
            issue_pk(k + 2, pk0, sp0)
            wait_scatter(rows0, dstb0, ss0)
            wait_pk(pk0, sp0)
            issue_gather(pk0, rows0, sr0)

        @pl.when(valid(k + 1))
        def _():
            wait_gather(pk1, rows1, sr1)
            compute(pk1, rows1, dstb1)
            issue_scatter(rows1, dstb1, ss1)

        @pl.when(valid(k + 3))
        def _():
            issue_pk(k + 3, pk1, sp1)

    wait_scatter(rows0, dstb0, ss0)
    wait_scatter(rows1, dstb1, ss1)

    plsc.subcore_barrier()
    _copy_out_clipped(acc, out_hbm, c, s)


def kernel(edge_index, edge_type, emb, W_rel1, W_root1, b1, W_rel2, W_root2, b2):
    src = edge_index[0].astype(jnp.int32)
    dst = edge_index[1].astype(jnp.int32)
    et = edge_type.astype(jnp.int32)
    gidx = et * N + src
    sidx = dst * R + et

    zeros_nd = jnp.zeros((ROWS_PER_SUB, DIM), jnp.float32)

    cnt2 = _count_pass(et, dst, zeros_nd)
    inv_nd = _inv_counts(cnt2)
    inv_c = inv_nd.reshape(N, R, 16)[:, :, 0].reshape(NR)
    scl = _scale_pass(sidx, inv_c)

    pk3 = jnp.stack(
        [gidx.reshape(NWIN, G),
         jax.lax.bitcast_convert_type(scl, jnp.int32).reshape(NWIN, G),
         dst.reshape(NWIN, G)], axis=1)
    pk = jnp.concatenate(
        [pk3, jnp.zeros((NWIN, 5, G), jnp.int32)], axis=1)

    h1 = _h_all(emb, W_rel1).reshape(R * N, DIM)
    acc1 = _edge_pass(pk, h1, zeros_nd)
    x1 = _combine(emb, W_root1, b1.reshape(1, DIM), acc1)

    h2 = _h_all(x1, W_rel2).reshape(R * N, DIM)
    acc2 = _edge_pass(pk, h2, zeros_nd)
    x2 = _combine(x1, W_root2, b2.reshape(1, DIM), acc2)
    return x2


# speedup vs baseline: 27.4901x; 1.2163x over previous
import dataclasses
import functools

import jax
import jax.numpy as jnp
from jax import lax
from jax.experimental import pallas as pl
from jax.experimental.pallas import tpu as pltpu
from jax.experimental.pallas import tpu_sc as plsc

N = 10000
R = 8
DIM = 128
E = 320000
G = 128
NWIN = E // G
NC = 2
NS = 16
NW = NC * NS
KMAX = (NWIN + NW - 1) // NW
NR = N * R
ROWS_PER_SUB = 640
ACC_ROWS = NS * ROWS_PER_SUB
CROWS_PER_SUB = NR // NS
NB = 2000

_mesh = plsc.VectorSubcoreMesh(core_axis_name="c", subcore_axis_name="s")

_sc_params = pltpu.CompilerParams()
if "needs_layout_passes" in pltpu.CompilerParams.__dataclass_fields__:
    _sc_params = dataclasses.replace(_sc_params, needs_layout_passes=False)


def _h_body(x_ref, w_ref, h_ref):
    h_ref[0] = jnp.dot(x_ref[...], w_ref[0], preferred_element_type=jnp.float32)


def _h_all(x, w_rel):
    return pl.pallas_call(
        _h_body,
        grid=(N // NB, R),
        in_specs=[
            pl.BlockSpec((NB, DIM), lambda i, r: (i, 0)),
            pl.BlockSpec((1, DIM, DIM), lambda i, r: (r, 0, 0)),
        ],
        out_specs=pl.BlockSpec((1, NB, DIM), lambda i, r: (r, i, 0)),
        out_shape=jax.ShapeDtypeStruct((R, N, DIM), jnp.float32),
    )(x, w_rel)


def _inv_body(c_ref, o_ref):
    o_ref[...] = 1.0 / jnp.maximum(c_ref[0] + c_ref[1], 1.0)


def _inv_counts(cnt2):
    return pl.pallas_call(
        _inv_body,
        grid=(N // NB,),
        in_specs=[pl.BlockSpec((2, NB, DIM), lambda i: (0, i, 0))],
        out_specs=pl.BlockSpec((NB, DIM), lambda i: (i, 0)),
        out_shape=jax.ShapeDtypeStruct((N, DIM), jnp.float32),
    )(cnt2)


def _combine_body(x_ref, w_ref, b_ref, a_ref, o_ref):
    y = jnp.dot(x_ref[...], w_ref[...], preferred_element_type=jnp.float32)
    o_ref[...] = jnp.maximum(y + b_ref[...] + a_ref[0] + a_ref[1], 0.0)


def _combine(x, w_root, b2d, acc):
    return pl.pallas_call(
        _combine_body,
        grid=(N // NB,),
        in_specs=[
            pl.BlockSpec((NB, DIM), lambda i: (i, 0)),
            pl.BlockSpec((DIM, DIM), lambda i: (0, 0)),
            pl.BlockSpec((1, DIM), lambda i: (0, 0)),
            pl.BlockSpec((2, NB, DIM), lambda i: (0, i, 0)),
        ],
        out_specs=pl.BlockSpec((NB, DIM), lambda i: (i, 0)),
        out_shape=jax.ShapeDtypeStruct((N, DIM), jnp.float32),
    )(x, w_root, b2d, acc)


def _copy_out_clipped(acc, out_hbm, c, s):
    @pl.when(s < NS - 1)
    def _():
        pltpu.sync_copy(acc.at[pl.ds(s * ROWS_PER_SUB, ROWS_PER_SUB)],
                        out_hbm.at[c, pl.ds(s * ROWS_PER_SUB, ROWS_PER_SUB)])

    @pl.when(s == NS - 1)
    def _():
        pltpu.sync_copy(
            acc.at[pl.ds((NS - 1) * ROWS_PER_SUB, N - (NS - 1) * ROWS_PER_SUB)],
            out_hbm.at[c, pl.ds((NS - 1) * ROWS_PER_SUB,
                                N - (NS - 1) * ROWS_PER_SUB)])


@functools.partial(
    pl.kernel,
    out_type=jax.ShapeDtypeStruct((NC, N, DIM), jnp.float32),
    mesh=_mesh,
    scratch_types=[
        pltpu.VMEM((8, G), jnp.int32),
        pltpu.VMEM((8, G), jnp.int32),
        pltpu.VMEM((G, DIM), jnp.float32),
        pltpu.VMEM((G, DIM), jnp.float32),
        pltpu.VMEM((1, G), jnp.int32),
        pltpu.VMEM((1, G), jnp.int32),
        pltpu.VMEM((1, G), jnp.int32),
        pltpu.VMEM((1, G), jnp.int32),
        pltpu.SemaphoreType.DMA,
        pltpu.SemaphoreType.DMA,
        pltpu.SemaphoreType.DMA,
        pltpu.SemaphoreType.DMA,
        pltpu.VMEM_SHARED((ACC_ROWS, DIM), jnp.float32),
    ],
    compiler_params=_sc_params,
)
def _count_pass(cpk_hbm, zeros_hbm, out_hbm,
                pk0, pk1, oh0, oh1, tb0, tb1, dstb0, dstb1,
                sp0, sp1, ss0, ss1, acc):
    c = lax.axis_index("c")
    s = lax.axis_index("s")
    wid = s * NC + c
    pltpu.sync_copy(zeros_hbm, acc.at[pl.ds(s * ROWS_PER_SUB, ROWS_PER_SUB)])
    pltpu.sync_copy(zeros_hbm.at[pl.ds(0, G)], oh0)
    pltpu.sync_copy(zeros_hbm.at[pl.ds(0, G)], oh1)
    plsc.subcore_barrier()

    ones16 = jnp.ones((16,), jnp.float32)
    zeros16f = jnp.zeros((16,), jnp.float32)
    lane16 = lax.iota(jnp.int32, 16)

    def valid(n):
        return (wid + NW * n) < NWIN

    def issue_pk(n, pk, sp):
        pltpu.async_copy(cpk_hbm.at[wid + NW * n], pk, sp)

    def wait_pk(pk, sp):
        pltpu.make_async_copy(cpk_hbm.at[0], pk, sp).wait()

    def build(pk, oh, tb, dstb, first):
        @pl.when(jnp.logical_not(first))
        def _():
            for m in range(G // 16):
                cols = tb[0, pl.ds(m * 16, 16)]
                rows16 = lane16 + m * 16
                for o in range(16):
                    plsc.store_scatter(oh, [rows16, cols + o], zeros16f)

        for m in range(G // 16):
            t16 = pk[0, pl.ds(m * 16, 16)]
            cols = t16 * 16
            rows16 = lane16 + m * 16
            tb[0, pl.ds(m * 16, 16)] = cols
            for o in range(16):
                plsc.store_scatter(oh, [rows16, cols + o], ones16)
        for m in range(G // 16):
            dstb[0, pl.ds(m * 16, 16)] = pk[1, pl.ds(m * 16, 16)]

    def issue_scatter(oh, dstb, ss):
        pltpu.async_copy(oh, acc.at[dstb.at[0]], ss, add=True)

    def wait_scatter(oh, dstb, ss):
        pltpu.make_async_copy(oh, acc.at[dstb.at[0]], ss).wait()

    @pl.when(valid(0))
    def _():
        issue_pk(0, pk0, sp0)

    @pl.when(valid(1))
    def _():
        issue_pk(1, pk1, sp1)

    @pl.loop(0, KMAX + 1, step=2)
    def _(k):
        @pl.when(valid(k))
        def _():
            wait_pk(pk0, sp0)

            @pl.when(k >= 2)
            def _():
                wait_scatter(oh0, dstb0, ss0)

            build(pk0, oh0, tb0, dstb0, k < 2)
            issue_scatter(oh0, dstb0, ss0)

        @pl.when(valid(k + 2))
        def _():
            issue_pk(k + 2, pk0, sp0)

        @pl.when(valid(k + 1))
        def _():
            wait_pk(pk1, sp1)

            @pl.when(k >= 2)
            def _():
                wait_scatter(oh1, dstb1, ss1)

            build(pk1, oh1, tb1, dstb1, k < 2)
            issue_scatter(oh1, dstb1, ss1)

        @pl.when(valid(k + 3))
        def _():
            issue_pk(k + 3, pk1, sp1)

    wait_scatter(oh0, dstb0, ss0)

    @pl.when(valid(1))
    def _():
        wait_scatter(oh1, dstb1, ss1)

    plsc.subcore_barrier()
    _copy_out_clipped(acc, out_hbm, c, s)


@functools.partial(
    pl.kernel,
    out_type=jax.ShapeDtypeStruct((E,), jnp.float32),
    mesh=_mesh,
    scratch_types=[
        pltpu.VMEM((1, G), jnp.int32),
        pltpu.VMEM((1, G), jnp.float32),
        pltpu.VMEM((NR,), jnp.float32),
    ],
    compiler_params=_sc_params,
)
def _scale_pass(sidx_hbm, inv_hbm, scl_hbm, sidx_v, scl_v, inv_v):
    c = lax.axis_index("c")
    s = lax.axis_index("s")
    wid = s * NC + c
    pltpu.sync_copy(inv_hbm, inv_v)

    @pl.loop(0, KMAX)
    def _(k):
        widx = wid + NW * k

        @pl.when(widx < NWIN)
        def _():
            base = widx * G
            pltpu.sync_copy(sidx_hbm.at[pl.ds(base, G)], sidx_v.at[0])

            for jj in range(G // 16):
                idxv = sidx_v[0, pl.ds(jj * 16, 16)]
                scl_v[0, pl.ds(jj * 16, 16)] = plsc.load_gather(inv_v, [idxv])

            pltpu.sync_copy(scl_v.at[0], scl_hbm.at[pl.ds(base, G)])


@functools.partial(
    pl.kernel,
    out_type=jax.ShapeDtypeStruct((NC, N, DIM), jnp.float32),
    mesh=_mesh,
    scratch_types=[
        pltpu.VMEM((8, G), jnp.int32),
        pltpu.VMEM((8, G), jnp.int32),
        pltpu.VMEM((G, DIM), jnp.float32),
        pltpu.VMEM((G, DIM), jnp.float32),
        pltpu.VMEM((1, G), jnp.int32),
        pltpu.VMEM((1, G), jnp.int32),
        pltpu.SemaphoreType.DMA,
        pltpu.SemaphoreType.DMA,
        pltpu.SemaphoreType.DMA,
        pltpu.SemaphoreType.DMA,
        pltpu.SemaphoreType.DMA,
        pltpu.SemaphoreType.DMA,
        pltpu.VMEM_SHARED((ACC_ROWS, DIM), jnp.float32),
    ],
    compiler_params=_sc_params,
)
def _edge_pass(pk_hbm, h_hbm, zeros_hbm, out_hbm,
               pk0, pk1, rows0, rows1, dstb0, dstb1,
               sp0, sp1, sr0, sr1, ss0, ss1, acc):
    c = lax.axis_index("c")
    s = lax.axis_index("s")
    wid = s * NC + c
    pltpu.sync_copy(zeros_hbm,
                    acc.at[pl.ds(s * ROWS_PER_SUB, ROWS_PER_SUB)])
    plsc.subcore_barrier()

    def valid(n):
        return (wid + NW * n) < NWIN

    def issue_pk(n, pk, sp):
        pltpu.async_copy(pk_hbm.at[wid + NW * n], pk, sp)

    def wait_pk(pk, sp):
        pltpu.make_async_copy(pk_hbm.at[0], pk, sp).wait()

    def issue_gather(pk, rows, sr):
        pltpu.async_copy(h_hbm.at[pk.at[0]], rows, sr)

    def wait_gather(pk, rows, sr):
        pltpu.make_async_copy(h_hbm.at[pk.at[0]], rows, sr).wait()

    def compute(pk, rows, dstb):
        @pl.loop(0, G)
        def _(j):
            jfull = jnp.full((16,), j, jnp.int32)
            sbits = plsc.load_gather(
                pk, [jnp.full((16,), 1, jnp.int32), jfull])
            sval = plsc.bitcast(sbits, jnp.float32)
            for cc in range(DIM // 16):
                rows[j, pl.ds(cc * 16, 16)] = (
                    rows[j, pl.ds(cc * 16, 16)] * sval)

        for cc in range(G // 16):
            dstb[0, pl.ds(cc * 16, 16)] = pk[2, pl.ds(cc * 16, 16)]

    def issue_scatter(rows, dstb, ss):
        pltpu.async_copy(rows, acc.at[dstb.at[0]], ss, add=True)

    def wait_scatter(rows, dstb, ss):
        pltpu.make_async_copy(rows, acc.at[dstb.at[0]], ss).wait()

    @pl.when(valid(0))
    def _():
        issue_pk(0, pk0, sp0)

    @pl.when(valid(1))
    def _():
        issue_pk(1, pk1, sp1)

    @pl.when(valid(0))
    def _():
        wait_pk(pk0, sp0)
        issue_gather(pk0, rows0, sr0)

    @pl.loop(0, KMAX + 1, step=2)
    def _(k):
        @pl.when(valid(k + 1))
        def _():
            wait_pk(pk1, sp1)

            @pl.when(k >= 1)
            def _():
                wait_scatter(rows1, dstb1, ss1)

            issue_gather(pk1, rows1, sr1)

        @pl.when(valid(k))
        def _():
            wait_gather(pk0, rows0, sr0)
            compute(pk0, rows0, dstb0)
            issue_scatter(rows0, dstb0, ss0)

        @pl.when(valid(k + 2))
        def _():
            issue_pk(k + 2, pk0, sp0)
            wait_scatter(rows0, dstb0, ss0)
            wait_pk(pk0, sp0)
            issue_gather(pk0, rows0, sr0)

        @pl.when(valid(k + 1))
        def _():
            wait_gather(pk1, rows1, sr1)
            compute(pk1, rows1, dstb1)
            issue_scatter(rows1, dstb1, ss1)

        @pl.when(valid(k + 3))
        def _():
            issue_pk(k + 3, pk1, sp1)

    wait_scatter(rows0, dstb0, ss0)
    wait_scatter(rows1, dstb1, ss1)

    plsc.subcore_barrier()
    _copy_out_clipped(acc, out_hbm, c, s)


def kernel(edge_index, edge_type, emb, W_rel1, W_root1, b1, W_rel2, W_root2, b2):
    src = edge_index[0].astype(jnp.int32)
    dst = edge_index[1].astype(jnp.int32)
    et = edge_type.astype(jnp.int32)
    gidx = et * N + src
    sidx = dst * R + et

    zeros_nd = jnp.zeros((ROWS_PER_SUB, DIM), jnp.float32)

    cpk = jnp.concatenate(
        [jnp.stack([et.reshape(NWIN, G), dst.reshape(NWIN, G)], axis=1),
         jnp.zeros((NWIN, 6, G), jnp.int32)], axis=1)

    cnt2 = _count_pass(cpk, zeros_nd)
    inv_nd = _inv_counts(cnt2)
    inv_c = inv_nd.reshape(N, R, 16)[:, :, 0].reshape(NR)
    scl = _scale_pass(sidx, inv_c)

    pk3 = jnp.stack(
        [gidx.reshape(NWIN, G),
         jax.lax.bitcast_convert_type(scl, jnp.int32).reshape(NWIN, G),
         dst.reshape(NWIN, G)], axis=1)
    pk = jnp.concatenate(
        [pk3, jnp.zeros((NWIN, 5, G), jnp.int32)], axis=1)

    h1 = _h_all(emb, W_rel1).reshape(R * N, DIM)
    acc1 = _edge_pass(pk, h1, zeros_nd)
    x1 = _combine(emb, W_root1, b1.reshape(1, DIM), acc1)

    h2 = _h_all(x1, W_rel2).reshape(R * N, DIM)
    acc2 = _edge_pass(pk, h2, zeros_nd)
    x2 = _combine(x1, W_root2, b2.reshape(1, DIM), acc2)
    return x2
